# baseline (speedup 1.0000x reference)
import jax
import jax.numpy as jnp
from jax import lax
from jax.experimental import pallas as pl
from jax.experimental.pallas import tpu as pltpu
from jax.experimental.pallas import tpu_sc as plsc

N_NODES = 10000
N_EDGES = 160000
D = 256
DH = D // 2

NC = 2
NS = 16
LANES = 16

E_PER_TILE = N_EDGES // NS
CHUNK = 104
N_FULL = E_PER_TILE // CHUNK
TAIL = E_PER_TILE - N_FULL * CHUNK
NROW = 3
NIDX = 6
WB_TILES = 10
WB_ROWS = N_NODES // WB_TILES


def _lane_bcast(wgrp, i):
  return lax.gather(
      wgrp, jnp.full((LANES, 1), i, jnp.int32),
      lax.GatherDimensionNumbers(offset_dims=(),
                                 collapsed_slice_dims=(0,),
                                 start_index_map=(0,)),
      slice_sizes=(1,), mode=lax.GatherScatterMode.PROMISE_IN_BOUNDS)


def _widen_scale_row(g_ref, f_ref, r, wb):
  for k in range(DH // 32):
    vi = g_ref[r, pl.ds(LANES * k, LANES)]
    lo = lax.bitcast_convert_type(vi << 16, jnp.float32)
    hi = lax.bitcast_convert_type(vi & jnp.full((LANES,), -65536, jnp.int32),
                                  jnp.float32)
    f_ref[r, pl.ds(32 * k, LANES)] = lo * wb
    f_ref[r, pl.ds(32 * k + LANES, LANES)] = hi * wb


def _scale_chunk(bf_ref, f_ref, w_ref, slot):
  def scale16(base, lanes):
    wgrp = w_ref[slot, pl.ds(base, LANES)]
    for i in lanes:
      _widen_scale_row(bf_ref, f_ref, base + i, _lane_bcast(wgrp, i))

  def scale_group(g, c2):
    scale16(g * LANES, range(LANES))
    return c2

  lax.fori_loop(0, CHUNK // LANES, scale_group, 0, unroll=False)
  if CHUNK % LANES:
    scale16(CHUNK - LANES, range(LANES - CHUNK % LANES, LANES))


def _spmm_body(x, src_h, dst_h, ew_h, zeros_h, out2,
               acc, sidx, didx, wv, gbuf0, gbuf1, fbuf0, fbuf1,
               tsidx, tdidx, twv, gs0, gs1, ss0, ss1, isem):
  cid = lax.axis_index("c")
  tid = lax.axis_index("s")
  ebase = tid * E_PER_TILE

  gbuf = (gbuf0, gbuf1)
  fbuf = (fbuf0, fbuf1)
  gsem = (gs0, gs1)
  ssem = (ss0, ss1)

  @pl.when(tid < WB_TILES)
  def _zero():
    pltpu.sync_copy(zeros_h, acc.at[pl.ds(tid * WB_ROWS, WB_ROWS)])

  plsc.subcore_barrier()

  def idx_load(c, slot):
    off = ebase + c * CHUNK
    pltpu.async_copy(src_h.at[pl.ds(off, CHUNK)], sidx.at[slot],
                     isem.at[slot])
    pltpu.async_copy(dst_h.at[pl.ds(off, CHUNK)], didx.at[slot],
                     isem.at[slot])
    pltpu.async_copy(ew_h.at[pl.ds(off, CHUNK)], wv.at[slot],
                     isem.at[slot])

  def idx_wait(c, slot):
    off = ebase + c * CHUNK
    pltpu.make_async_copy(src_h.at[pl.ds(off, CHUNK)], sidx.at[slot],
                          isem.at[slot]).wait()
    pltpu.make_async_copy(dst_h.at[pl.ds(off, CHUNK)], didx.at[slot],
                          isem.at[slot]).wait()
    pltpu.make_async_copy(ew_h.at[pl.ds(off, CHUNK)], wv.at[slot],
                          isem.at[slot]).wait()

  def gather_start(slot, b):
    pltpu.async_copy(x.at[cid].at[sidx.at[slot]], gbuf[b], gsem[b])

  def gather_wait(slot, b):
    pltpu.make_async_copy(x.at[cid].at[sidx.at[slot]], gbuf[b],
                          gsem[b]).wait()

  def scatter_start(slot, b):
    pltpu.async_copy(fbuf[b], acc.at[didx.at[slot]], ssem[b], add=True)

  def scatter_wait(slot, b):
    pltpu.make_async_copy(fbuf[b], acc.at[didx.at[slot]],
                          ssem[b]).wait()

  for c in range(3):
    idx_load(c, c)
  idx_wait(0, 0)
  gather_start(0, 0)

  def super_step(jj, carry):
    for u in range(6):
      j = jj + u
      b = u % 2
      s = u

      gather_wait(s, b)

      @pl.when(j + 1 < N_FULL)
      def _g():
        idx_wait(j + 1, (s + 1) % 6)
        gather_start((s + 1) % 6, 1 - b)

      @pl.when(j >= 2)
      def _ws():
        scatter_wait((s + 4) % 6, b)

      _scale_chunk(gbuf[b], fbuf[b], wv, s)
      scatter_start(s, b)

      @pl.when(j + 3 < N_FULL)
      def _pf():
        idx_load(j + 3, (s + 3) % 6)

    return carry

  lax.fori_loop(0, N_FULL // 6, lambda i, c: super_step(i * 6, c), 0,
                unroll=False)

  scatter_wait(4, 0)
  scatter_wait(5, 1)

  toff = ebase + N_FULL * CHUNK
  pltpu.sync_copy(src_h.at[pl.ds(toff, TAIL)], tsidx)
  pltpu.sync_copy(dst_h.at[pl.ds(toff, TAIL)], tdidx)
  pltpu.sync_copy(ew_h.at[pl.ds(toff, TAIL)], twv)
  pltpu.async_copy(x.at[cid].at[tsidx], gbuf0.at[pl.ds(0, TAIL)],
                   gs0).wait()

  wgrp = twv[...]
  for i in range(TAIL):
    _widen_scale_row(gbuf0, fbuf0, i, _lane_bcast(wgrp, i))

  pltpu.async_copy(fbuf0.at[pl.ds(0, TAIL)], acc.at[tdidx], ss0,
                   add=True).wait()

  plsc.subcore_barrier()

  @pl.when(tid < WB_TILES)
  def _writeback():
    row0 = tid * WB_ROWS
    pltpu.sync_copy(acc.at[pl.ds(row0, WB_ROWS)],
                    out2.at[cid].at[pl.ds(row0, WB_ROWS)])


@jax.jit
def _spmm(x, src, dst, ew, zeros):
  mesh = plsc.VectorSubcoreMesh(core_axis_name="c", subcore_axis_name="s")
  f = pl.kernel(
      _spmm_body,
      out_type=jax.ShapeDtypeStruct((NC, N_NODES, DH), jnp.float32),
      mesh=mesh,
      compiler_params=pltpu.CompilerParams(use_tc_tiling_on_sc=False),
      scratch_types=[
          pltpu.VMEM_SHARED((N_NODES, DH), jnp.float32),
          pltpu.VMEM((NIDX, CHUNK), jnp.int32),
          pltpu.VMEM((NIDX, CHUNK), jnp.int32),
          pltpu.VMEM((NIDX, CHUNK), jnp.float32),
          pltpu.VMEM((CHUNK, DH // 2), jnp.int32),
          pltpu.VMEM((CHUNK, DH // 2), jnp.int32),
          pltpu.VMEM((CHUNK, DH), jnp.float32),
          pltpu.VMEM((CHUNK, DH), jnp.float32),
          pltpu.VMEM((TAIL,), jnp.int32),
          pltpu.VMEM((TAIL,), jnp.int32),
          pltpu.VMEM((TAIL,), jnp.float32),
          pltpu.SemaphoreType.DMA,
          pltpu.SemaphoreType.DMA,
          pltpu.SemaphoreType.DMA,
          pltpu.SemaphoreType.DMA,
          pltpu.SemaphoreType.DMA((NIDX,)),
      ],
  )
  return f(x, src, dst, ew, zeros)


def _dense_body(hia_ref, hib_ref, h0_ref, w_ref, s_ref, out_ref):
  a = s_ref[0]
  th = s_ref[1]
  h0 = h0_ref[...]
  sa = (1.0 - a) * hia_ref[0] + a * h0[:, :DH]
  sb = (1.0 - a) * hib_ref[0] + a * h0[:, DH:]
  mm = jnp.dot(sa, w_ref[:DH, :], preferred_element_type=jnp.float32) \
      + jnp.dot(sb, w_ref[DH:, :], preferred_element_type=jnp.float32)
  out_ref[:, :DH] = th * mm[:, :DH] + (1.0 - th) * sa
  out_ref[:, DH:] = th * mm[:, DH:] + (1.0 - th) * sb


@jax.jit
def _dense(hi2, h0, W, scal):
  BM = 1000
  return pl.pallas_call(
      _dense_body,
      grid=(N_NODES // BM,),
      in_specs=[
          pl.BlockSpec((1, BM, DH), lambda i: (0, i, 0)),
          pl.BlockSpec((1, BM, DH), lambda i: (1, i, 0)),
          pl.BlockSpec((BM, D), lambda i: (i, 0)),
          pl.BlockSpec((D, D), lambda i: (0, 0)),
          pl.BlockSpec(memory_space=pltpu.SMEM),
      ],
      out_specs=pl.BlockSpec((BM, D), lambda i: (i, 0)),
      out_shape=jax.ShapeDtypeStruct((N_NODES, D), jnp.float32),
  )(hi2, hi2, h0, W, scal)


_PERM = []
for _k in range(D // 32):
  for _m in range(32):
    _PERM.append(32 * _k + (_m // 2 if _m % 2 == 0 else 16 + _m // 2))
_PERM = tuple(_PERM)


def kernel(x, edge_index, edge_weight, h0, W, lamda, alpha, l):
  src = edge_index[1].astype(jnp.int32)
  dst = edge_index[0].astype(jnp.int32)
  ew = edge_weight.astype(jnp.float32)
  zeros = jnp.zeros((WB_ROWS, DH), jnp.float32)
  xb = x[:, jnp.array(_PERM, jnp.int32)].astype(jnp.bfloat16)
  xi = lax.bitcast_convert_type(xb.reshape(N_NODES, D // 2, 2), jnp.int32)
  xi2 = jnp.stack([xi[:, :DH // 2], xi[:, DH // 2:]])

  hi2 = _spmm(xi2, src, dst, ew, zeros)

  theta = jnp.log(lamda / l + 1.0).astype(jnp.float32)
  scal = jnp.stack([alpha.astype(jnp.float32), theta])
  return _dense(hi2, h0, W, scal)


# speedup vs baseline: 2.1638x; 2.1638x over previous
import jax
import jax.numpy as jnp
from jax import lax
from jax.experimental import pallas as pl
from jax.experimental.pallas import tpu as pltpu
from jax.experimental.pallas import tpu_sc as plsc

N_NODES = 10000
N_EDGES = 160000
D = 256
DH = D // 2

NC = 2
NS = 16
LANES = 16

E_PER_TILE = N_EDGES // NS
CHUNK = 104
N_FULL = E_PER_TILE // CHUNK
TAIL = E_PER_TILE - N_FULL * CHUNK
NROW = 3
NIDX = 6
WB_TILES = 10
WB_ROWS = N_NODES // WB_TILES


def _lane_bcast(wgrp, i):
  return lax.gather(
      wgrp, jnp.full((LANES, 1), i, jnp.int32),
      lax.GatherDimensionNumbers(offset_dims=(),
                                 collapsed_slice_dims=(0,),
                                 start_index_map=(0,)),
      slice_sizes=(1,), mode=lax.GatherScatterMode.PROMISE_IN_BOUNDS)


def _scale_chunk(rows_ref, w_ref, slot):
  def scale16(base, lanes):
    wgrp = w_ref[slot, pl.ds(base, LANES)]
    for i in lanes:
      wb = _lane_bcast(wgrp, i)
      r = base + i
      for k in range(DH // LANES):
        sl = pl.ds(k * LANES, LANES)
        rows_ref[r, sl] = rows_ref[r, sl] * wb

  def scale_group(g, c2):
    scale16(g * LANES, range(LANES))
    return c2

  lax.fori_loop(0, CHUNK // LANES, scale_group, 0, unroll=False)
  if CHUNK % LANES:
    scale16(CHUNK - LANES, range(LANES - CHUNK % LANES, LANES))


def _spmm_body(x, src_h, dst_h, ew_h, zeros_h, out2,
               acc, sidx, didx, wv, rows0, rows1, rows2,
               tsidx, tdidx, twv, gs0, gs1, gs2, ss0, ss1, ss2, isem):
  cid = lax.axis_index("c")
  tid = lax.axis_index("s")
  ebase = tid * E_PER_TILE
  col = cid * DH

  rows = (rows0, rows1, rows2)
  gsem = (gs0, gs1, gs2)
  ssem = (ss0, ss1, ss2)

  @pl.when(tid < WB_TILES)
  def _zero():
    pltpu.sync_copy(zeros_h, acc.at[pl.ds(tid * WB_ROWS, WB_ROWS)])

  plsc.subcore_barrier()

  def idx_load(c, slot):
    off = ebase + c * CHUNK
    pltpu.async_copy(src_h.at[pl.ds(off, CHUNK)], sidx.at[slot],
                     isem.at[slot])
    pltpu.async_copy(dst_h.at[pl.ds(off, CHUNK)], didx.at[slot],
                     isem.at[slot])
    pltpu.async_copy(ew_h.at[pl.ds(off, CHUNK)], wv.at[slot],
                     isem.at[slot])

  def idx_wait(c, slot):
    off = ebase + c * CHUNK
    pltpu.make_async_copy(src_h.at[pl.ds(off, CHUNK)], sidx.at[slot],
                          isem.at[slot]).wait()
    pltpu.make_async_copy(dst_h.at[pl.ds(off, CHUNK)], didx.at[slot],
                          isem.at[slot]).wait()
    pltpu.make_async_copy(ew_h.at[pl.ds(off, CHUNK)], wv.at[slot],
                          isem.at[slot]).wait()

  def gather_start(slot, b):
    pltpu.async_copy(x.at[cid].at[sidx.at[slot]], rows[b], gsem[b])

  def gather_wait(slot, b):
    pltpu.make_async_copy(x.at[cid].at[sidx.at[slot]], rows[b],
                          gsem[b]).wait()

  def scatter_start(slot, b):
    pltpu.async_copy(rows[b], acc.at[didx.at[slot]], ssem[b], add=True)

  def scatter_wait(slot, b):
    pltpu.make_async_copy(rows[b], acc.at[didx.at[slot]],
                          ssem[b]).wait()

  for c in range(3):
    idx_load(c, c)
  idx_wait(0, 0)
  gather_start(0, 0)

  def super_step(jj, carry):
    for u in range(6):
      j = jj + u
      b = u % 3
      s = u

      gather_wait(s, b)

      @pl.when(j + 1 < N_FULL)
      def _g():
        idx_wait(j + 1, (s + 1) % 6)
        gather_start((s + 1) % 6, (u + 1) % 3)

      _scale_chunk(rows[b], wv, s)

      @pl.when(j >= 1)
      def _ws():
        scatter_wait((s + 5) % 6, (u + 2) % 3)

      scatter_start(s, b)

      @pl.when(j + 3 < N_FULL)
      def _pf():
        idx_load(j + 3, (s + 3) % 6)

    return carry

  lax.fori_loop(0, N_FULL // 6, lambda i, c: super_step(i * 6, c), 0,
                unroll=False)

  scatter_wait(5, 2)

  toff = ebase + N_FULL * CHUNK
  pltpu.sync_copy(src_h.at[pl.ds(toff, TAIL)], tsidx)
  pltpu.sync_copy(dst_h.at[pl.ds(toff, TAIL)], tdidx)
  pltpu.sync_copy(ew_h.at[pl.ds(toff, TAIL)], twv)
  tgt = rows0.at[pl.ds(0, TAIL)]
  pltpu.async_copy(x.at[cid].at[tsidx], tgt, gs0).wait()

  wgrp = twv[...]
  for i in range(TAIL):
    wb = _lane_bcast(wgrp, i)
    for k in range(DH // LANES):
      sl = pl.ds(k * LANES, LANES)
      rows0[i, sl] = rows0[i, sl] * wb

  pltpu.async_copy(tgt, acc.at[tdidx], ss0, add=True).wait()

  plsc.subcore_barrier()

  @pl.when(tid < WB_TILES)
  def _writeback():
    row0 = tid * WB_ROWS
    pltpu.sync_copy(acc.at[pl.ds(row0, WB_ROWS)],
                    out2.at[cid].at[pl.ds(row0, WB_ROWS)])


@jax.jit
def _spmm(x, src, dst, ew, zeros):
  mesh = plsc.VectorSubcoreMesh(core_axis_name="c", subcore_axis_name="s")
  f = pl.kernel(
      _spmm_body,
      out_type=jax.ShapeDtypeStruct((NC, N_NODES, DH), jnp.float32),
      mesh=mesh,
      scratch_types=[
          pltpu.VMEM_SHARED((N_NODES, DH), jnp.float32),
          pltpu.VMEM((NIDX, CHUNK), jnp.int32),
          pltpu.VMEM((NIDX, CHUNK), jnp.int32),
          pltpu.VMEM((NIDX, CHUNK), jnp.float32),
          pltpu.VMEM((CHUNK, DH), jnp.float32),
          pltpu.VMEM((CHUNK, DH), jnp.float32),
          pltpu.VMEM((CHUNK, DH), jnp.float32),
          pltpu.VMEM((TAIL,), jnp.int32),
          pltpu.VMEM((TAIL,), jnp.int32),
          pltpu.VMEM((TAIL,), jnp.float32),
          pltpu.SemaphoreType.DMA,
          pltpu.SemaphoreType.DMA,
          pltpu.SemaphoreType.DMA,
          pltpu.SemaphoreType.DMA,
          pltpu.SemaphoreType.DMA,
          pltpu.SemaphoreType.DMA,
          pltpu.SemaphoreType.DMA((NIDX,)),
      ],
  )
  return f(x, src, dst, ew, zeros)


def _dense_body(hia_ref, hib_ref, h0_ref, w_ref, s_ref, out_ref):
  a = s_ref[0]
  th = s_ref[1]
  h0 = h0_ref[...]
  sa = (1.0 - a) * hia_ref[0] + a * h0[:, :DH]
  sb = (1.0 - a) * hib_ref[0] + a * h0[:, DH:]
  mm = jnp.dot(sa, w_ref[:DH, :], preferred_element_type=jnp.float32) \
      + jnp.dot(sb, w_ref[DH:, :], preferred_element_type=jnp.float32)
  out_ref[:, :DH] = th * mm[:, :DH] + (1.0 - th) * sa
  out_ref[:, DH:] = th * mm[:, DH:] + (1.0 - th) * sb


@jax.jit
def _dense(hi2, h0, W, scal):
  BM = 1000
  return pl.pallas_call(
      _dense_body,
      grid=(N_NODES // BM,),
      in_specs=[
          pl.BlockSpec((1, BM, DH), lambda i: (0, i, 0)),
          pl.BlockSpec((1, BM, DH), lambda i: (1, i, 0)),
          pl.BlockSpec((BM, D), lambda i: (i, 0)),
          pl.BlockSpec((D, D), lambda i: (0, 0)),
          pl.BlockSpec(memory_space=pltpu.SMEM),
      ],
      out_specs=pl.BlockSpec((BM, D), lambda i: (i, 0)),
      out_shape=jax.ShapeDtypeStruct((N_NODES, D), jnp.float32),
  )(hi2, hi2, h0, W, scal)


def kernel(x, edge_index, edge_weight, h0, W, lamda, alpha, l):
  src = edge_index[1].astype(jnp.int32)
  dst = edge_index[0].astype(jnp.int32)
  ew = edge_weight.astype(jnp.float32)
  zeros = jnp.zeros((WB_ROWS, DH), jnp.float32)
  x2 = jnp.stack([x[:, :DH], x[:, DH:]])

  hi2 = _spmm(x2, src, dst, ew, zeros)

  theta = jnp.log(lamda / l + 1.0).astype(jnp.float32)
  scal = jnp.stack([alpha.astype(jnp.float32), theta])
  return _dense(hi2, h0, W, scal)
